# baseline (speedup 1.0000x reference)
import functools

import jax
import jax.numpy as jnp
from jax import lax
from jax.experimental import pallas as pl
from jax.experimental.pallas import tpu as pltpu
from jax.experimental.pallas import tpu_sc as plsc

VOCAB = 256
DIM = 768
LANES = 16
NC = 2
NS = 16
NW = NC * NS
NIB = 4
NRB = 2


@functools.lru_cache(maxsize=None)
def _make_emb(batch: int, seq: int):
    IPW = batch // NW
    T0 = seq // 8 * 8
    NTAIL = seq - T0
    TCOL = T0 + 8
    NCOL = TCOL + 8
    mesh = plsc.VectorSubcoreMesh(core_axis_name="c", subcore_axis_name="s")

    @functools.partial(
        pl.kernel,
        mesh=mesh,
        out_type=jax.ShapeDtypeStruct((batch, seq, DIM), jnp.float32),
        scratch_types=[
            pltpu.VMEM((NIB, NCOL), jnp.int32),
            pltpu.VMEM((NRB, seq, DIM), jnp.float32),
            pltpu.VMEM((8, DIM), jnp.float32),
        ] + [pltpu.SemaphoreType.DMA] * (NIB + 2 * NRB + 1),
        compiler_params=pltpu.CompilerParams(use_tc_tiling_on_sc=True),
    )
    def emb(ids_hbm, table_hbm, out_hbm, idx_v, rows_v, tail_v, *sems):
        isem = sems[:NIB]
        gsem = sems[NIB:NIB + NRB]
        wsem = sems[NIB + NRB:NIB + 2 * NRB]
        tsem = sems[NIB + 2 * NRB]
        wid = lax.axis_index("s") * NC + lax.axis_index("c")
        base = wid * IPW

        def idx_load(j, ib):
            return pltpu.make_async_copy(
                ids_hbm.at[base + j], idx_v.at[ib], isem[ib])

        def gather_main(ib, rb):
            return pltpu.make_async_copy(
                table_hbm.at[idx_v.at[ib, pl.ds(0, T0)]],
                rows_v.at[rb, pl.ds(0, T0)], gsem[rb])

        def gather_tail(ib):
            return pltpu.make_async_copy(
                table_hbm.at[idx_v.at[ib, pl.ds(TCOL, 8)]], tail_v, tsem)

        def write(j, rb):
            return pltpu.make_async_copy(
                rows_v.at[rb], out_hbm.at[base + j], wsem[rb])

        def fix_tail(rb):
            def move(c, carry):
                off = c * LANES
                for s in range(NTAIL):
                    rows_v[rb, T0 + s, pl.ds(off, LANES)] = (
                        tail_v[s, pl.ds(off, LANES)])
                return carry
            lax.fori_loop(0, DIM // LANES, move, 0)

        for k in range(NIB):
            idx_load(k, k).start()
        idx_load(0, 0).wait()
        gather_main(0, 0).start()
        gather_tail(0).start()
        idx_load(0, 1).wait()
        gather_main(1, 1).start()

        def body(i, carry):
            for k in range(NIB):
                j = i * NIB + k
                rb = k % NRB
                gather_main(k, rb).wait()
                gather_tail(k).wait()
                fix_tail(rb)

                @pl.when(j + 1 < IPW)
                def _next_tail():
                    gather_tail((k + 1) % NIB).start()

                write(j, rb).start()

                @pl.when(j + NIB < IPW)
                def _prefetch_ids():
                    idx_load(j + NIB, k).start()

                @pl.when(j + NRB < IPW)
                def _next_gather():
                    write(j, rb).wait()
                    idx_load(0, (k + NRB) % NIB).wait()
                    gather_main((k + NRB) % NIB, rb).start()
            return carry

        lax.fori_loop(0, IPW // NIB, body, 0)
        for rb in range(NRB):
            write(0, rb).wait()

    return emb


def kernel(token_ids, table):
    batch, seq = token_ids.shape
    ids = token_ids.astype(jnp.int32)
    t0 = seq // 8 * 8
    zeros3 = jnp.zeros((batch, t0 + 8 - seq), jnp.int32)
    ids_packed = jnp.concatenate(
        [ids, zeros3, ids[:, t0:], zeros3], axis=1)
    return _make_emb(batch, seq)(ids_packed, table)


# speedup vs baseline: 1.0040x; 1.0040x over previous
import functools

import jax
import jax.numpy as jnp
from jax import lax
from jax.experimental import pallas as pl
from jax.experimental.pallas import tpu as pltpu
from jax.experimental.pallas import tpu_sc as plsc

VOCAB = 256
DIM = 768
NC = 2
NS = 16
NW = NC * NS
NIB = 4
NRB = 2


@functools.lru_cache(maxsize=None)
def _make_emb(batch: int, seq: int):
    IPW = batch // NW
    T0 = seq // 8 * 8
    mesh = plsc.VectorSubcoreMesh(core_axis_name="c", subcore_axis_name="s")

    @functools.partial(
        pl.kernel,
        mesh=mesh,
        out_type=jax.ShapeDtypeStruct((batch, seq, DIM), jnp.float32),
        scratch_types=[
            pltpu.VMEM((NIB, T0), jnp.int32),
            pltpu.VMEM((NIB, 8), jnp.int32),
            pltpu.VMEM((NRB, T0, DIM), jnp.float32),
            pltpu.VMEM((NRB, 8, DIM), jnp.float32),
        ] + [pltpu.SemaphoreType.DMA] * (2 * NIB + 4 * NRB),
        compiler_params=pltpu.CompilerParams(use_tc_tiling_on_sc=True),
    )
    def emb(ids_hbm, tids_hbm, table_hbm, out_hbm,
            idx_v, tidx_v, rows_v, tail_v, *sems):
        isem = sems[:NIB]
        itsem = sems[NIB:2 * NIB]
        gsem = sems[2 * NIB:2 * NIB + NRB]
        tsem = sems[2 * NIB + NRB:2 * NIB + 2 * NRB]
        wasem = sems[2 * NIB + 2 * NRB:2 * NIB + 3 * NRB]
        wbsem = sems[2 * NIB + 3 * NRB:]
        wid = lax.axis_index("s") * NC + lax.axis_index("c")
        base = wid * IPW
        t0_dyn = pl.multiple_of(lax.axis_index("c") * 0 + T0, 8)

        def idx_load(j, ib):
            return pltpu.make_async_copy(
                ids_hbm.at[base + j], idx_v.at[ib], isem[ib])

        def tidx_load(j, ib):
            return pltpu.make_async_copy(
                tids_hbm.at[base + j], tidx_v.at[ib], itsem[ib])

        def gather_main(ib, rb):
            return pltpu.make_async_copy(
                table_hbm.at[idx_v.at[ib]], rows_v.at[rb], gsem[rb])

        def gather_tail(ib, tb):
            return pltpu.make_async_copy(
                table_hbm.at[tidx_v.at[ib]], tail_v.at[tb], tsem[tb])

        def write_main(j, rb):
            return pltpu.make_async_copy(
                rows_v.at[rb], out_hbm.at[base + j, pl.ds(0, T0)], wasem[rb])

        def write_tail(j, tb):
            return pltpu.make_async_copy(
                tail_v.at[tb], out_hbm.at[base + j, pl.ds(t0_dyn, 8)],
                wbsem[tb])

        for k in range(NIB):
            idx_load(k, k).start()
            tidx_load(k, k).start()
        for k in range(NRB):
            idx_load(0, k).wait()
            tidx_load(0, k).wait()
            gather_main(k, k).start()
            gather_tail(k, k).start()

        def body(i, carry):
            for k in range(NIB):
                j = i * NIB + k
                rb = k % NRB
                gather_main(k, rb).wait()
                gather_tail(k, rb).wait()
                write_main(j, rb).start()
                write_tail(j, rb).start()

                @pl.when(j + NIB < IPW)
                def _prefetch_ids():
                    idx_load(j + NIB, k).start()
                    tidx_load(j + NIB, k).start()

                @pl.when(j + NRB < IPW)
                def _next_gather():
                    write_main(j, rb).wait()
                    write_tail(j, rb).wait()
                    kn = (k + NRB) % NIB
                    idx_load(0, kn).wait()
                    tidx_load(0, kn).wait()
                    gather_main(kn, rb).start()
                    gather_tail(kn, rb).start()
            return carry

        lax.fori_loop(0, IPW // NIB, body, 0)
        for rb in range(NRB):
            write_main(0, rb).wait()
            write_tail(0, rb).wait()

    return emb


def kernel(token_ids, table):
    batch, seq = token_ids.shape
    ids = token_ids.astype(jnp.int32)
    t0 = seq // 8 * 8
    ids_main = ids[:, :t0]
    ids_tail = jnp.pad(ids[:, t0:], ((0, 0), (0, t0 + 8 - seq)))
    return _make_emb(batch, seq)(ids_main, ids_tail, table)
